# Initial kernel scaffold; baseline (speedup 1.0000x reference)
import jax
import jax.numpy as jnp
from jax.experimental import pallas as pl


def kernel(x, Wg, bg, W1, b1, W2, b2):
    raise NotImplementedError("write your pallas kernel here")


# speedup vs baseline: 3.3247x; 3.3247x over previous
import functools
import math

import jax
import jax.numpy as jnp
from jax.experimental import pallas as pl
from jax.experimental.pallas import tpu as pltpu

NEG_INF = -1e30


def _moe_dense_body(x_ref, wg_ref, bg_ref, w1_ref, b1_ref, w2_ref, b2_ref,
                    out_ref, gate_ref, *, num_experts):
    k = pl.program_id(1)
    xb = x_ref[...]

    @pl.when(k == 0)
    def _compute_gates():
        scores = jnp.dot(xb, wg_ref[...],
                         preferred_element_type=jnp.float32) + bg_ref[...]
        iota = jax.lax.broadcasted_iota(jnp.int32, scores.shape, 1)
        m0 = jnp.max(scores, axis=-1, keepdims=True)
        i0 = jnp.min(jnp.where(scores == m0, iota, num_experts),
                     axis=-1, keepdims=True)
        masked = jnp.where(iota == i0, NEG_INF, scores)
        m1 = jnp.max(masked, axis=-1, keepdims=True)
        i1 = jnp.min(jnp.where(masked == m1, iota, num_experts),
                     axis=-1, keepdims=True)
        g0 = 1.0 / (1.0 + jnp.exp(m1 - m0))
        gate_ref[...] = (jnp.where(iota == i0, g0, 0.0)
                         + jnp.where(iota == i1, 1.0 - g0, 0.0))

    h = jnp.dot(xb, w1_ref[0], preferred_element_type=jnp.float32) + b1_ref[0]
    h = 0.5 * h * (1.0 + jax.lax.erf(h * (1.0 / math.sqrt(2.0))))
    y = jnp.dot(h, w2_ref[0], preferred_element_type=jnp.float32) + b2_ref[0]
    gates = gate_ref[...]
    col = jax.lax.broadcasted_iota(jnp.int32, gates.shape, 1)
    gate_k = jnp.sum(jnp.where(col == k, gates, 0.0), axis=1, keepdims=True)
    contrib = gate_k * y

    @pl.when(k == 0)
    def _init():
        out_ref[...] = contrib

    @pl.when(k != 0)
    def _acc():
        out_ref[...] += contrib


def kernel(x, Wg, bg, W1, b1, W2, b2):
    B, N, EMB = x.shape
    T = B * N
    E, _, HID = W1.shape
    TT = min(1024, T)
    NI = T // TT
    xf = x.reshape(T, EMB)

    out = pl.pallas_call(
        functools.partial(_moe_dense_body, num_experts=E),
        grid=(NI, E),
        in_specs=[
            pl.BlockSpec((TT, EMB), lambda i, k: (i, 0)),
            pl.BlockSpec((EMB, E), lambda i, k: (0, 0)),
            pl.BlockSpec((1, E), lambda i, k: (0, 0)),
            pl.BlockSpec((1, EMB, HID), lambda i, k: (k, 0, 0)),
            pl.BlockSpec((1, 1, HID), lambda i, k: (k, 0, 0)),
            pl.BlockSpec((1, HID, EMB), lambda i, k: (k, 0, 0)),
            pl.BlockSpec((1, 1, EMB), lambda i, k: (k, 0, 0)),
        ],
        out_specs=pl.BlockSpec((TT, EMB), lambda i, k: (i, 0)),
        out_shape=jax.ShapeDtypeStruct((T, EMB), jnp.float32),
        scratch_shapes=[pltpu.VMEM((TT, E), jnp.float32)],
    )(xf, Wg, bg.reshape(1, E), W1, b1.reshape(E, 1, HID), W2,
      b2.reshape(E, 1, EMB))
    return out.reshape(B, N, EMB)


# speedup vs baseline: 3.3378x; 1.0039x over previous
import functools
import math

import jax
import jax.numpy as jnp
from jax.experimental import pallas as pl
from jax.experimental.pallas import tpu as pltpu

NEG_INF = -1e30


def _moe_dense_body(x_ref, wg_ref, bg_ref, w1_ref, b1_ref, w2_ref, b2_ref,
                    out_ref, gate_ref, *, num_experts):
    k = pl.program_id(1)
    xb = x_ref[...]

    @pl.when(k == 0)
    def _compute_gates():
        scores = jnp.dot(xb, wg_ref[...],
                         preferred_element_type=jnp.float32) + bg_ref[...]
        iota = jax.lax.broadcasted_iota(jnp.int32, scores.shape, 1)
        m0 = jnp.max(scores, axis=-1, keepdims=True)
        i0 = jnp.min(jnp.where(scores == m0, iota, num_experts),
                     axis=-1, keepdims=True)
        masked = jnp.where(iota == i0, NEG_INF, scores)
        m1 = jnp.max(masked, axis=-1, keepdims=True)
        i1 = jnp.min(jnp.where(masked == m1, iota, num_experts),
                     axis=-1, keepdims=True)
        g0 = 1.0 / (1.0 + jnp.exp(m1 - m0))
        gate_ref[...] = (jnp.where(iota == i0, g0, 0.0)
                         + jnp.where(iota == i1, 1.0 - g0, 0.0))

    h = jnp.dot(xb.astype(jnp.bfloat16), w1_ref[0].astype(jnp.bfloat16),
                preferred_element_type=jnp.float32) + b1_ref[0]
    h = 0.5 * h * (1.0 + jax.lax.erf(h * (1.0 / math.sqrt(2.0))))
    y = jnp.dot(h.astype(jnp.bfloat16), w2_ref[0].astype(jnp.bfloat16),
                preferred_element_type=jnp.float32) + b2_ref[0]
    gates = gate_ref[...]
    col = jax.lax.broadcasted_iota(jnp.int32, gates.shape, 1)
    gate_k = jnp.sum(jnp.where(col == k, gates, 0.0), axis=1, keepdims=True)
    contrib = gate_k * y

    @pl.when(k == 0)
    def _init():
        out_ref[...] = contrib

    @pl.when(k != 0)
    def _acc():
        out_ref[...] += contrib


def kernel(x, Wg, bg, W1, b1, W2, b2):
    B, N, EMB = x.shape
    T = B * N
    E, _, HID = W1.shape
    TT = min(1024, T)
    NI = T // TT
    xf = x.reshape(T, EMB)

    out = pl.pallas_call(
        functools.partial(_moe_dense_body, num_experts=E),
        grid=(NI, E),
        in_specs=[
            pl.BlockSpec((TT, EMB), lambda i, k: (i, 0)),
            pl.BlockSpec((EMB, E), lambda i, k: (0, 0)),
            pl.BlockSpec((1, E), lambda i, k: (0, 0)),
            pl.BlockSpec((1, EMB, HID), lambda i, k: (k, 0, 0)),
            pl.BlockSpec((1, 1, HID), lambda i, k: (k, 0, 0)),
            pl.BlockSpec((1, HID, EMB), lambda i, k: (k, 0, 0)),
            pl.BlockSpec((1, 1, EMB), lambda i, k: (k, 0, 0)),
        ],
        out_specs=pl.BlockSpec((TT, EMB), lambda i, k: (i, 0)),
        out_shape=jax.ShapeDtypeStruct((T, EMB), jnp.float32),
        scratch_shapes=[pltpu.VMEM((TT, E), jnp.float32)],
    )(xf, Wg, bg.reshape(1, E), W1, b1.reshape(E, 1, HID), W2,
      b2.reshape(E, 1, EMB))
    return out.reshape(B, N, EMB)
